# Initial kernel scaffold; baseline (speedup 1.0000x reference)
import functools

import jax
import jax.numpy as jnp
from jax import lax
from jax.experimental import pallas as pl
from jax.experimental.pallas import tpu as pltpu
from jax.experimental.pallas import tpu_sc as plsc

_BATCH = 16384
_HIST = 50
_VOCAB = 1_000_000
_DIM = 32

_NW = 32
_RPW = _BATCH // _NW
_IPW = _RPW * _HIST
_IDX_ROWS = _IPW // 128

_TC_BLK = 8192


def _matvec_body(emb_ref, w_ref, b_ref, t_ref):
    acc = jnp.dot(emb_ref[...], w_ref[...], preferred_element_type=jnp.float32)
    t_ref[...] = (acc + b_ref[...]) * (1.0 / _HIST)


def _build_table(emb_table, W, b2):
    grid = pl.cdiv(_VOCAB, _TC_BLK)
    return pl.pallas_call(
        _matvec_body,
        grid=(grid,),
        in_specs=[
            pl.BlockSpec((_TC_BLK, _DIM), lambda i: (i, 0)),
            pl.BlockSpec((_DIM, 1), lambda i: (0, 0)),
            pl.BlockSpec((1, 1), lambda i: (0, 0)),
        ],
        out_specs=pl.BlockSpec((_TC_BLK, 1), lambda i: (i, 0)),
        out_shape=jax.ShapeDtypeStruct((_VOCAB, 1), jnp.float32),
    )(emb_table, W, b2)


def _sc_pool_body(xt_hbm, t_hbm, out_hbm, idx_v, vals_v, out_v, sem):
    wid = lax.axis_index("s") * 2 + lax.axis_index("c")
    pltpu.sync_copy(xt_hbm.at[wid], idx_v)
    pltpu.async_copy(t_hbm.at[idx_v], vals_v, sem).wait()
    for g in range(_RPW // 16):
        r0, col = g // 8, 16 * (g % 8)
        acc = vals_v[r0, pl.ds(col, 16)]
        for l in range(1, _HIST):
            acc = acc + vals_v[4 * l + r0, pl.ds(col, 16)]
        out_v[pl.ds(16 * g, 16)] = 1.0 / (1.0 + jnp.exp(-acc))
    pltpu.sync_copy(out_v, out_hbm.at[pl.ds(wid * _RPW, _RPW)])


_sc_pool = functools.partial(
    pl.kernel,
    out_type=jax.ShapeDtypeStruct((_BATCH,), jnp.float32),
    mesh=plsc.VectorSubcoreMesh(core_axis_name="c", subcore_axis_name="s"),
    scratch_types=[
        pltpu.VMEM((_IDX_ROWS, 128), jnp.int32),
        pltpu.VMEM((_IDX_ROWS, 128), jnp.float32),
        pltpu.VMEM((_RPW,), jnp.float32),
        pltpu.SemaphoreType.DMA,
    ],
)(_sc_pool_body)


def kernel(x, emb_table, W, b):
    t2 = _build_table(emb_table, W, b.reshape(1, 1))
    xt = (
        x.reshape(_NW, _RPW, _HIST)
        .transpose(0, 2, 1)
        .reshape(_NW, _IDX_ROWS, 128)
    )
    y = _sc_pool(xt, t2.reshape(-1))
    return y.reshape(_BATCH, 1)


# speedup vs baseline: 2.0512x; 2.0512x over previous
import functools

import jax
import jax.numpy as jnp
from jax import lax
from jax.experimental import pallas as pl
from jax.experimental.pallas import tpu as pltpu
from jax.experimental.pallas import tpu_sc as plsc

_BATCH = 16384
_HIST = 50
_VOCAB = 1_000_000
_DIM = 32

_NW = 32
_RPW = _BATCH // _NW
_IPW = _RPW * _HIST
_IDX_ROWS = _IPW // 128

_TC_BLK = 8192


def _matvec_body(emb_ref, w_ref, b_ref, t_ref):
    acc = jnp.dot(emb_ref[...], w_ref[...], preferred_element_type=jnp.float32)
    t_ref[...] = (acc + b_ref[...]) * (1.0 / _HIST)


def _build_table(emb_table, W, b2):
    grid = pl.cdiv(_VOCAB, _TC_BLK)
    return pl.pallas_call(
        _matvec_body,
        grid=(grid,),
        in_specs=[
            pl.BlockSpec((_TC_BLK, _DIM), lambda i: (i, 0)),
            pl.BlockSpec((_DIM, 1), lambda i: (0, 0)),
            pl.BlockSpec((1, 1), lambda i: (0, 0)),
        ],
        out_specs=pl.BlockSpec((_TC_BLK, 1), lambda i: (i, 0)),
        out_shape=jax.ShapeDtypeStruct((_VOCAB, 1), jnp.float32),
    )(emb_table, W, b2)


def _sc_pool_body(xt_hbm, t_hbm, out_hbm, idx_v, vals_v, out_v, sem):
    wid = lax.axis_index("s") * 2 + lax.axis_index("c")
    pltpu.sync_copy(xt_hbm.at[wid], idx_v)
    pltpu.async_copy(t_hbm.at[idx_v], vals_v, sem).wait()
    for g in range(_RPW // 16):
        acc = vals_v[pl.ds(16 * g, 16)]
        for l in range(1, _HIST):
            acc = acc + vals_v[pl.ds(512 * l + 16 * g, 16)]
        out_v[pl.ds(16 * g, 16)] = 1.0 / (1.0 + jnp.exp(-acc))
    pltpu.sync_copy(out_v, out_hbm.at[pl.ds(wid * _RPW, _RPW)])


_sc_pool = functools.partial(
    pl.kernel,
    out_type=jax.ShapeDtypeStruct((_BATCH,), jnp.float32),
    mesh=plsc.VectorSubcoreMesh(core_axis_name="c", subcore_axis_name="s"),
    scratch_types=[
        pltpu.VMEM((_IPW,), jnp.int32),
        pltpu.VMEM((_IPW,), jnp.float32),
        pltpu.VMEM((_RPW,), jnp.float32),
        pltpu.SemaphoreType.DMA,
    ],
)(_sc_pool_body)


def kernel(x, emb_table, W, b):
    t2 = _build_table(emb_table, W, b.reshape(1, 1))
    xt = (
        x.reshape(_NW, _RPW, _HIST)
        .transpose(0, 2, 1)
        .reshape(_NW, _IPW)
    )
    y = _sc_pool(xt, t2.reshape(-1))
    return y.reshape(_BATCH, 1)


# speedup vs baseline: 3.2032x; 1.5617x over previous
import functools

import jax
import jax.numpy as jnp
from jax import lax
from jax.experimental import pallas as pl
from jax.experimental.pallas import tpu as pltpu
from jax.experimental.pallas import tpu_sc as plsc

_BATCH = 16384
_HIST = 50
_VOCAB = 1_000_000
_DIM = 32

_NW = 32
_RPW = _BATCH // _NW
_IPW = _RPW * _HIST
_IDX_ROWS = _IPW // 128

_TC_BLK = 8192
_TC_GRID = pl.cdiv(_VOCAB, _TC_BLK)
_VOCAB_PAD = _TC_GRID * _TC_BLK


def _matvec_body(emb_ref, wt_ref, b_ref, t_ref):
    emb_t = jax.lax.transpose(emb_ref[...], (1, 0))
    acc = jnp.dot(wt_ref[...], emb_t, preferred_element_type=jnp.float32)
    t_ref[...] = (acc + b_ref[...]) * (1.0 / _HIST)


def _build_table(emb_table, Wt, b2):
    return pl.pallas_call(
        _matvec_body,
        grid=(_TC_GRID,),
        in_specs=[
            pl.BlockSpec((_TC_BLK, _DIM), lambda i: (i, 0)),
            pl.BlockSpec((1, _DIM), lambda i: (0, 0)),
            pl.BlockSpec((1, 1), lambda i: (0, 0)),
        ],
        out_specs=pl.BlockSpec((1, _TC_BLK), lambda i: (0, i)),
        out_shape=jax.ShapeDtypeStruct((1, _VOCAB_PAD), jnp.float32),
    )(emb_table, Wt, b2)


def _sc_pool_body(xt_hbm, t_hbm, out_hbm, idx_v, vals_v, out_v, sem):
    wid = lax.axis_index("s") * 2 + lax.axis_index("c")
    pltpu.sync_copy(xt_hbm.at[wid], idx_v)
    pltpu.async_copy(t_hbm.at[idx_v], vals_v, sem).wait()
    for g in range(_RPW // 16):
        acc = vals_v[pl.ds(16 * g, 16)]
        for l in range(1, _HIST):
            acc = acc + vals_v[pl.ds(512 * l + 16 * g, 16)]
        out_v[pl.ds(16 * g, 16)] = 1.0 / (1.0 + jnp.exp(-acc))
    pltpu.sync_copy(out_v, out_hbm.at[pl.ds(wid * _RPW, _RPW)])


_sc_pool = functools.partial(
    pl.kernel,
    out_type=jax.ShapeDtypeStruct((_BATCH,), jnp.float32),
    mesh=plsc.VectorSubcoreMesh(core_axis_name="c", subcore_axis_name="s"),
    scratch_types=[
        pltpu.VMEM((_IPW,), jnp.int32),
        pltpu.VMEM((_IPW,), jnp.float32),
        pltpu.VMEM((_RPW,), jnp.float32),
        pltpu.SemaphoreType.DMA,
    ],
)(_sc_pool_body)


def kernel(x, emb_table, W, b):
    t2 = _build_table(emb_table, W.reshape(1, _DIM), b.reshape(1, 1))
    xt = (
        x.reshape(_NW, _RPW, _HIST)
        .transpose(0, 2, 1)
        .reshape(_NW, _IPW)
    )
    y = _sc_pool(xt, t2.reshape(-1))
    return y.reshape(_BATCH, 1)
